# baseline (speedup 1.0000x reference)
import functools

import jax
import jax.numpy as jnp
import numpy as np
from jax import lax
from jax.experimental import pallas as pl
from jax.experimental.pallas import tpu as pltpu
from jax.experimental.pallas import tpu_sc as plsc

V = 128
D = 64
DD = D * D
DERTA = 10.0
L = 16
NC = 2
NS = 16
NW = NC * NS
EPR = V // NW

_Pa = np.zeros((D, DD), dtype=np.float32)
_Pb = np.zeros((D, DD), dtype=np.float32)
for _a in range(D):
    _Pa[_a, _a * D:(_a + 1) * D] = 1.0
for _b in range(D):
    _Pb[_b, _b::D] = 1.0

BI = 8


def _prep_kernel(feat_ref, pa_ref, pb_ref, a_ref, b_ref):
    feats = feat_ref[...]
    a_ref[...] = jnp.dot(feats, pa_ref[...], preferred_element_type=jnp.float32)
    b_ref[...] = jnp.dot(feats, pb_ref[...], preferred_element_type=jnp.float32)


def _edge_attr_sc(cos_hbm, eij_hbm, sij_hbm, cos_v, e_v, s_v, sem):
    wid = lax.axis_index("s") * NC + lax.axis_index("c")
    i0 = wid * EPR
    pltpu.async_copy(cos_hbm.at[pl.ds(i0, EPR)], cos_v, sem).wait()
    for r in range(EPR):
        for k in range(V // L):
            sl = pl.ds(k * L, L)
            e = 0.5 * (1.0 - cos_v[r, sl])
            e_v[r, sl] = e
            s_v[r, sl] = jnp.exp(-DERTA * e)
    h1 = pltpu.async_copy(e_v, eij_hbm.at[pl.ds(i0, EPR)], sem)
    h2 = pltpu.async_copy(s_v, sij_hbm.at[pl.ds(i0, EPR)], sem)
    h1.wait()
    h2.wait()


_edge_attr_call = functools.partial(
    pl.kernel,
    out_type=[
        jax.ShapeDtypeStruct((V, V), jnp.float32),
        jax.ShapeDtypeStruct((V, V), jnp.float32),
    ],
    mesh=plsc.VectorSubcoreMesh(core_axis_name="c", subcore_axis_name="s"),
    scratch_types=[
        pltpu.VMEM((EPR, V), jnp.float32),
        pltpu.VMEM((EPR, V), jnp.float32),
        pltpu.VMEM((EPR, V), jnp.float32),
        pltpu.SemaphoreType.DMA,
    ],
)(_edge_attr_sc)


NBUF = 2
NSTEPS = V // BI


def _stream_kernel(a_ref, b_ref, c_hbm, scratch, sems):
    i = pl.program_id(0)
    s = jax.lax.rem(i, NBUF)

    @pl.when(i >= NBUF)
    def _():
        pltpu.make_async_copy(
            scratch.at[s],
            c_hbm.at[pl.ds((i - NBUF) * BI, BI)],
            sems.at[s],
        ).wait()

    scratch[s] = a_ref[0][:, None, :] * b_ref[...][None, :, :]
    pltpu.make_async_copy(
        scratch.at[s],
        c_hbm.at[pl.ds(i * BI, BI)],
        sems.at[s],
    ).start()

    @pl.when(i == NSTEPS - 1)
    def _():
        for dj in range(NBUF):
            j = NSTEPS - NBUF + dj
            pltpu.make_async_copy(
                scratch.at[j % NBUF],
                c_hbm.at[pl.ds(j * BI, BI)],
                sems.at[j % NBUF],
            ).wait()


@jax.jit
def kernel(costheta, features):
    eij, sij = _edge_attr_call(costheta)

    a_full, b_full = pl.pallas_call(
        _prep_kernel,
        out_shape=[
            jax.ShapeDtypeStruct((V, DD), jnp.float32),
            jax.ShapeDtypeStruct((V, DD), jnp.float32),
        ],
    )(features, _Pa, _Pb)

    c_flat = pl.pallas_call(
        _stream_kernel,
        grid=(V // BI,),
        in_specs=[
            pl.BlockSpec((1, BI, DD), lambda i: (i, 0, 0)),
            pl.BlockSpec((V, DD), lambda i: (0, 0)),
        ],
        out_specs=pl.BlockSpec(memory_space=pl.ANY),
        out_shape=jax.ShapeDtypeStruct((V, V, DD), jnp.float32),
        scratch_shapes=[
            pltpu.VMEM((NBUF, BI, V, DD), jnp.float32),
            pltpu.SemaphoreType.DMA((NBUF,)),
        ],
    )(a_full.reshape(V // BI, BI, DD), b_full)
    return (eij, sij, c_flat.reshape(V, V, D, D))


# speedup vs baseline: 1.0013x; 1.0013x over previous
import functools

import jax
import jax.numpy as jnp
import numpy as np
from jax import lax
from jax.experimental import pallas as pl
from jax.experimental.pallas import tpu as pltpu
from jax.experimental.pallas import tpu_sc as plsc

V = 128
D = 64
DD = D * D
DERTA = 10.0
L = 16
NC = 2
NS = 16
NW = NC * NS
EPR = V // NW

_Pa = np.zeros((D, DD), dtype=np.float32)
_Pb = np.zeros((D, DD), dtype=np.float32)
for _a in range(D):
    _Pa[_a, _a * D:(_a + 1) * D] = 1.0
for _b in range(D):
    _Pb[_b, _b::D] = 1.0

BI = 8


def _prep_kernel(feat_ref, pa_ref, pb_ref, a_ref, b_ref):
    feats = feat_ref[...]
    a_ref[...] = jnp.dot(feats, pa_ref[...], preferred_element_type=jnp.float32)
    b_ref[...] = jnp.dot(feats, pb_ref[...], preferred_element_type=jnp.float32)


def _edge_attr_sc(cos_hbm, eij_hbm, sij_hbm, cos_v, e_v, s_v, sem):
    wid = lax.axis_index("s") * NC + lax.axis_index("c")
    i0 = wid * EPR
    pltpu.async_copy(cos_hbm.at[pl.ds(i0, EPR)], cos_v, sem).wait()
    for r in range(EPR):
        for k in range(V // L):
            sl = pl.ds(k * L, L)
            e = 0.5 * (1.0 - cos_v[r, sl])
            e_v[r, sl] = e
            s_v[r, sl] = jnp.exp(-DERTA * e)
    h1 = pltpu.async_copy(e_v, eij_hbm.at[pl.ds(i0, EPR)], sem)
    h2 = pltpu.async_copy(s_v, sij_hbm.at[pl.ds(i0, EPR)], sem)
    h1.wait()
    h2.wait()


def _edge_attr_call(costheta):
    call = functools.partial(
        pl.kernel,
        out_type=[
            jax.ShapeDtypeStruct((V, V), jnp.float32),
            jax.ShapeDtypeStruct((V, V), jnp.float32),
        ],
        mesh=plsc.VectorSubcoreMesh(core_axis_name="c", subcore_axis_name="s"),
        scratch_types=[
            pltpu.VMEM((EPR, V), jnp.float32),
            pltpu.VMEM((EPR, V), jnp.float32),
            pltpu.VMEM((EPR, V), jnp.float32),
            pltpu.SemaphoreType.DMA,
        ],
    )(_edge_attr_sc)
    return call(costheta)


NBUF = 2
NSTEPS = V // BI


def _stream_kernel(a_ref, b_ref, c_hbm, scratch, sems):
    i = pl.program_id(0)
    s = jax.lax.rem(i, NBUF)

    @pl.when(i >= NBUF)
    def _():
        pltpu.make_async_copy(
            scratch.at[s],
            c_hbm.at[pl.ds((i - NBUF) * BI, BI)],
            sems.at[s],
        ).wait()

    scratch[s] = a_ref[0][:, None, :] * b_ref[...][None, :, :]
    pltpu.make_async_copy(
        scratch.at[s],
        c_hbm.at[pl.ds(i * BI, BI)],
        sems.at[s],
    ).start()

    @pl.when(i == NSTEPS - 1)
    def _():
        for dj in range(NBUF):
            j = NSTEPS - NBUF + dj
            pltpu.make_async_copy(
                scratch.at[j % NBUF],
                c_hbm.at[pl.ds(j * BI, BI)],
                sems.at[j % NBUF],
            ).wait()


@jax.jit
def kernel(costheta, features):
    eij, sij = _edge_attr_call(costheta)

    a_full, b_full = pl.pallas_call(
        _prep_kernel,
        out_shape=[
            jax.ShapeDtypeStruct((V, DD), jnp.float32),
            jax.ShapeDtypeStruct((V, DD), jnp.float32),
        ],
    )(features, _Pa, _Pb)

    c_flat = pl.pallas_call(
        _stream_kernel,
        grid=(V // BI,),
        in_specs=[
            pl.BlockSpec((1, BI, DD), lambda i: (i, 0, 0)),
            pl.BlockSpec((V, DD), lambda i: (0, 0)),
        ],
        out_specs=pl.BlockSpec(memory_space=pl.ANY),
        out_shape=jax.ShapeDtypeStruct((V, V, DD), jnp.float32),
        scratch_shapes=[
            pltpu.VMEM((NBUF, BI, V, DD), jnp.float32),
            pltpu.SemaphoreType.DMA((NBUF,)),
        ],
    )(a_full.reshape(V // BI, BI, DD), b_full)
    return (eij, sij, c_flat.reshape(V, V, D, D))
